# baseline (speedup 1.0000x reference)
import functools

import jax
import jax.numpy as jnp
from jax.experimental import pallas as pl
from jax.experimental.pallas import tpu as pltpu


def _fused_kernel(x1_hbm, x2_hbm,
                  w1_ref, b1_ref, w2_ref, b2_ref,
                  wc1a_ref, wc1b_ref, bc1_ref, wc2_ref, bc2_ref,
                  o1_hbm, o2_hbm,
                  s1_sc, s2_sc, d1_sc, d2_sc, in_sem, out_sem,
                  *, n_b, inv_hw, lambda_c, lambda_s):
    c = pl.program_id(0)
    base = c * n_b
    C = s1_sc.shape[1]
    bf16 = jnp.bfloat16

    def fetch(slot, i):
        pltpu.make_async_copy(x1_hbm.at[base + i], s1_sc.at[slot],
                              in_sem.at[slot, 0]).start()
        pltpu.make_async_copy(x2_hbm.at[base + i], s2_sc.at[slot],
                              in_sem.at[slot, 1]).start()

    def fetch_wait(slot):
        pltpu.make_async_copy(x1_hbm.at[base], s1_sc.at[slot],
                              in_sem.at[slot, 0]).wait()
        pltpu.make_async_copy(x2_hbm.at[base], s2_sc.at[slot],
                              in_sem.at[slot, 1]).wait()

    def put(slot, i):
        pltpu.make_async_copy(d1_sc.at[slot], o1_hbm.at[base + i],
                              out_sem.at[slot, 0]).start()
        pltpu.make_async_copy(d2_sc.at[slot], o2_hbm.at[base + i],
                              out_sem.at[slot, 1]).start()

    def put_wait(slot):
        pltpu.make_async_copy(d1_sc.at[slot], o1_hbm.at[base],
                              out_sem.at[slot, 0]).wait()
        pltpu.make_async_copy(d2_sc.at[slot], o2_hbm.at[base],
                              out_sem.at[slot, 1]).wait()

    fetch(0, 0)
    if n_b > 1:
        fetch(1, 1)

    def body(i, _):
        slot = jax.lax.rem(i, 2)
        fetch_wait(slot)
        x1 = s1_sc[slot]
        x2 = s2_sc[slot]

        avg1 = jnp.sum(x1, axis=1, keepdims=True) * inv_hw
        avg2 = jnp.sum(x2, axis=1, keepdims=True) * inv_hw
        max1 = jnp.max(x1, axis=1, keepdims=True)
        max2 = jnp.max(x2, axis=1, keepdims=True)
        y = jnp.concatenate([avg1, avg2, max1, max2], axis=0)
        h = (jnp.dot(w1_ref[...], y.astype(bf16),
                     preferred_element_type=jnp.float32)
             + b1_ref[...])
        h = jnp.maximum(h, 0.0)
        z = jax.nn.sigmoid(
            jnp.dot(w2_ref[...], h.astype(bf16),
                    preferred_element_type=jnp.float32)
            + b2_ref[...])
        cw0 = z[0:C]
        cw1 = z[C:2 * C]

        hs = (jnp.dot(wc1a_ref[...], x1.astype(bf16),
                      preferred_element_type=jnp.float32)
              + jnp.dot(wc1b_ref[...], x2.astype(bf16),
                        preferred_element_type=jnp.float32)
              + bc1_ref[...])
        hs = jnp.maximum(hs, 0.0)
        s = jax.nn.sigmoid(
            jnp.dot(wc2_ref[...], hs.astype(bf16),
                    preferred_element_type=jnp.float32)
            + bc2_ref[...])
        s0 = s[0:1]
        s1 = s[1:2]

        @pl.when(i >= 2)
        def _reclaim():
            put_wait(slot)
        d1_sc[slot] = x1 + lambda_c * (cw1 * x2) + lambda_s * (s1 * x2)
        d2_sc[slot] = x2 + lambda_c * (cw0 * x1) + lambda_s * (s0 * x1)
        put(slot, i)

        @pl.when(i + 2 < n_b)
        def _prefetch():
            fetch(slot, i + 2)
        return ()

    jax.lax.fori_loop(0, n_b, body, ())
    put_wait(jax.lax.rem(jnp.int32(max(n_b - 2, 0)), 2))
    if n_b > 1:
        put_wait(jax.lax.rem(jnp.int32(n_b - 1), 2))


def kernel(x1, x2, w1, b1, w2, b2, wc1, bc1, wc2, bc2):
    B, C, H, W = x1.shape
    HW = H * W
    lambda_c = 0.5
    lambda_s = 0.5
    bf16 = jnp.bfloat16
    x1r = x1.reshape(B, C, HW)
    x2r = x2.reshape(B, C, HW)

    n_c = 2 if B % 2 == 0 else 1
    n_b = B // n_c

    hid_c = w1.shape[1]
    w1t = w1.T.astype(bf16)
    b1c = b1.reshape(hid_c, 1)

    w2t = w2.T.astype(bf16)
    b2c = b2.reshape(2 * C, 1)

    hid_s = wc1.shape[1]
    wc1a = wc1[0:C, :].T.astype(bf16)
    wc1b = wc1[C:2 * C, :].T.astype(bf16)
    bc1c = bc1.reshape(hid_s, 1)

    wc2t = wc2.T.astype(bf16)
    bc2c = bc2.reshape(2, 1)

    hbm = pl.BlockSpec(memory_space=pltpu.MemorySpace.HBM)

    def const2d(shape):
        return pl.BlockSpec(shape, lambda c: (0, 0))

    o1, o2 = pl.pallas_call(
        functools.partial(_fused_kernel, n_b=n_b, inv_hw=1.0 / HW,
                          lambda_c=lambda_c, lambda_s=lambda_s),
        out_shape=(jax.ShapeDtypeStruct((B, C, HW), x1.dtype),
                   jax.ShapeDtypeStruct((B, C, HW), x1.dtype)),
        grid_spec=pltpu.PrefetchScalarGridSpec(
            num_scalar_prefetch=0,
            grid=(n_c,),
            in_specs=[
                hbm, hbm,
                const2d((hid_c, 4 * C)), const2d((hid_c, 1)),
                const2d((2 * C, hid_c)), const2d((2 * C, 1)),
                const2d((hid_s, C)), const2d((hid_s, C)),
                const2d((hid_s, 1)),
                const2d((2, hid_s)), const2d((2, 1)),
            ],
            out_specs=[hbm, hbm],
            scratch_shapes=[
                pltpu.VMEM((2, C, HW), jnp.float32),
                pltpu.VMEM((2, C, HW), jnp.float32),
                pltpu.VMEM((2, C, HW), jnp.float32),
                pltpu.VMEM((2, C, HW), jnp.float32),
                pltpu.SemaphoreType.DMA((2, 2)),
                pltpu.SemaphoreType.DMA((2, 2)),
            ],
        ),
        compiler_params=pltpu.CompilerParams(
            dimension_semantics=("parallel",)),
    )(x1r, x2r, w1t, b1c, w2t, b2c, wc1a, wc1b, bc1c, wc2t, bc2c)

    return o1.reshape(B, C, H, W), o2.reshape(B, C, H, W)


# speedup vs baseline: 1.0431x; 1.0431x over previous
import functools

import jax
import jax.numpy as jnp
from jax.experimental import pallas as pl
from jax.experimental.pallas import tpu as pltpu


def _fused_kernel(x1_ref, x2_ref,
                  w1_ref, b1_ref, w2_ref, b2_ref,
                  wc1a_ref, wc1b_ref, bc1_ref, wc2_ref, bc2_ref,
                  o1_ref, o2_ref, *, lambda_c, lambda_s):
    x1 = x1_ref[0]
    x2 = x2_ref[0]
    C = x1.shape[0]
    bf16 = jnp.bfloat16

    def _fold(a, op):
        w = a.shape[1]
        while w > 128:
            w //= 2
            a = op(a[:, :w], a[:, w:2 * w])
        return a

    sum1 = jnp.sum(_fold(x1, jnp.add), axis=1, keepdims=True)
    sum2 = jnp.sum(_fold(x2, jnp.add), axis=1, keepdims=True)
    max1 = jnp.max(_fold(x1, jnp.maximum), axis=1, keepdims=True)
    max2 = jnp.max(_fold(x2, jnp.maximum), axis=1, keepdims=True)
    y = jnp.concatenate([sum1, sum2, max1, max2], axis=0)
    h = (jnp.dot(w1_ref[...], y.astype(bf16),
                 preferred_element_type=jnp.float32)
         + b1_ref[...])
    h = jnp.maximum(h, 0.0)
    z = jax.nn.sigmoid(
        jnp.dot(w2_ref[...], h.astype(bf16),
                preferred_element_type=jnp.float32)
        + b2_ref[...])
    cw0 = lambda_c * z[0:C]
    cw1 = lambda_c * z[C:2 * C]

    hs = (jnp.dot(wc1a_ref[...], x1.astype(bf16),
                  preferred_element_type=jnp.float32)
          + jnp.dot(wc1b_ref[...], x2.astype(bf16),
                    preferred_element_type=jnp.float32)
          + bc1_ref[...])
    hs = jnp.maximum(hs, 0.0).astype(bf16)
    s = jax.nn.sigmoid(
        jnp.dot(wc2_ref[...], hs, preferred_element_type=jnp.float32)
        + bc2_ref[...])
    s0 = lambda_s * s[0:1]
    s1 = lambda_s * s[1:2]

    g1 = cw1 + s1
    g2 = cw0 + s0
    o1_ref[0] = x1 + g1 * x2
    o2_ref[0] = x2 + g2 * x1


def kernel(x1, x2, w1, b1, w2, b2, wc1, bc1, wc2, bc2):
    B, C, H, W = x1.shape
    HW = H * W
    lambda_c = 0.5
    lambda_s = 0.5
    bf16 = jnp.bfloat16
    x1r = x1.reshape(B, C, HW)
    x2r = x2.reshape(B, C, HW)

    hid_c = w1.shape[1]
    scale = jnp.concatenate([jnp.full((2 * C, 1), 1.0 / HW, jnp.float32),
                             jnp.ones((2 * C, 1), jnp.float32)], axis=0)
    w1t = (w1 * scale).T.astype(bf16)
    b1c = b1.reshape(hid_c, 1)

    w2t = w2.T.astype(bf16)
    b2c = b2.reshape(2 * C, 1)

    hid_s = wc1.shape[1]
    wc1a = wc1[0:C, :].T.astype(bf16)
    wc1b = wc1[C:2 * C, :].T.astype(bf16)
    bc1c = bc1.reshape(hid_s, 1)

    wc2t = wc2.T.astype(bf16)
    bc2c = bc2.reshape(2, 1)

    img_spec = pl.BlockSpec((1, C, HW), lambda b: (b, 0, 0))

    def const2d(shape):
        return pl.BlockSpec(shape, lambda b: (0, 0))

    o1, o2 = pl.pallas_call(
        functools.partial(_fused_kernel,
                          lambda_c=lambda_c, lambda_s=lambda_s),
        out_shape=(jax.ShapeDtypeStruct((B, C, HW), x1.dtype),
                   jax.ShapeDtypeStruct((B, C, HW), x1.dtype)),
        grid=(B,),
        in_specs=[
            img_spec, img_spec,
            const2d((hid_c, 4 * C)), const2d((hid_c, 1)),
            const2d((2 * C, hid_c)), const2d((2 * C, 1)),
            const2d((hid_s, C)), const2d((hid_s, C)), const2d((hid_s, 1)),
            const2d((2, hid_s)), const2d((2, 1)),
        ],
        out_specs=[img_spec, img_spec],
        compiler_params=pltpu.CompilerParams(
            dimension_semantics=("parallel",)),
    )(x1r, x2r, w1t, b1c, w2t, b2c, wc1a, wc1b, bc1c, wc2t, bc2c)

    return o1.reshape(B, C, H, W), o2.reshape(B, C, H, W)


# speedup vs baseline: 1.0761x; 1.0316x over previous
import functools

import jax
import jax.numpy as jnp
from jax.experimental import pallas as pl
from jax.experimental.pallas import tpu as pltpu

_BATCH_PER_STEP = 2


def _fused_kernel(x1_ref, x2_ref,
                  w1_ref, b1_ref, w2_ref, b2_ref,
                  wc1a_ref, wc1b_ref, bc1_ref, wc2_ref, bc2_ref,
                  o1_ref, o2_ref, *, n_j, lambda_c, lambda_s):
    C = x1_ref.shape[1]
    bf16 = jnp.bfloat16

    def _fold(a, op):
        w = a.shape[1]
        while w > 128:
            w //= 2
            a = op(a[:, :w], a[:, w:2 * w])
        return a

    for j in range(n_j):
        x1 = x1_ref[j]
        x2 = x2_ref[j]

        sum1 = jnp.sum(_fold(x1, jnp.add), axis=1, keepdims=True)
        sum2 = jnp.sum(_fold(x2, jnp.add), axis=1, keepdims=True)
        max1 = jnp.max(_fold(x1, jnp.maximum), axis=1, keepdims=True)
        max2 = jnp.max(_fold(x2, jnp.maximum), axis=1, keepdims=True)
        y = jnp.concatenate([sum1, sum2, max1, max2], axis=0)
        h = (jnp.dot(w1_ref[...], y.astype(bf16),
                     preferred_element_type=jnp.float32)
             + b1_ref[...])
        h = jnp.maximum(h, 0.0)
        z = jax.nn.sigmoid(
            jnp.dot(w2_ref[...], h.astype(bf16),
                    preferred_element_type=jnp.float32)
            + b2_ref[...])
        cw0 = lambda_c * z[0:C]
        cw1 = lambda_c * z[C:2 * C]

        hs = (jnp.dot(wc1a_ref[...], x1.astype(bf16),
                      preferred_element_type=jnp.float32)
              + jnp.dot(wc1b_ref[...], x2.astype(bf16),
                        preferred_element_type=jnp.float32)
              + bc1_ref[...])
        hs = jnp.maximum(hs, 0.0).astype(bf16)
        s = jax.nn.sigmoid(
            jnp.dot(wc2_ref[...], hs, preferred_element_type=jnp.float32)
            + bc2_ref[...])
        s0 = lambda_s * s[0:1]
        s1 = lambda_s * s[1:2]

        o1_ref[j] = x1 + (cw1 + s1) * x2
        o2_ref[j] = x2 + (cw0 + s0) * x1


def kernel(x1, x2, w1, b1, w2, b2, wc1, bc1, wc2, bc2):
    B, C, H, W = x1.shape
    HW = H * W
    lambda_c = 0.5
    lambda_s = 0.5
    bf16 = jnp.bfloat16
    x1r = x1.reshape(B, C, HW)
    x2r = x2.reshape(B, C, HW)

    n_j = _BATCH_PER_STEP if B % _BATCH_PER_STEP == 0 else 1
    n_steps = B // n_j

    hid_c = w1.shape[1]
    scale = jnp.concatenate([jnp.full((2 * C, 1), 1.0 / HW, jnp.float32),
                             jnp.ones((2 * C, 1), jnp.float32)], axis=0)
    w1t = (w1 * scale).T.astype(bf16)
    b1c = b1.reshape(hid_c, 1)

    w2t = w2.T.astype(bf16)
    b2c = b2.reshape(2 * C, 1)

    hid_s = wc1.shape[1]
    wc1a = wc1[0:C, :].T.astype(bf16)
    wc1b = wc1[C:2 * C, :].T.astype(bf16)
    bc1c = bc1.reshape(hid_s, 1)

    wc2t = wc2.T.astype(bf16)
    bc2c = bc2.reshape(2, 1)

    img_spec = pl.BlockSpec((n_j, C, HW), lambda b: (b, 0, 0))

    def const2d(shape):
        return pl.BlockSpec(shape, lambda b: (0, 0))

    o1, o2 = pl.pallas_call(
        functools.partial(_fused_kernel, n_j=n_j,
                          lambda_c=lambda_c, lambda_s=lambda_s),
        out_shape=(jax.ShapeDtypeStruct((B, C, HW), x1.dtype),
                   jax.ShapeDtypeStruct((B, C, HW), x1.dtype)),
        grid=(n_steps,),
        in_specs=[
            img_spec, img_spec,
            const2d((hid_c, 4 * C)), const2d((hid_c, 1)),
            const2d((2 * C, hid_c)), const2d((2 * C, 1)),
            const2d((hid_s, C)), const2d((hid_s, C)), const2d((hid_s, 1)),
            const2d((2, hid_s)), const2d((2, 1)),
        ],
        out_specs=[img_spec, img_spec],
        compiler_params=pltpu.CompilerParams(
            dimension_semantics=("parallel",)),
    )(x1r, x2r, w1t, b1c, w2t, b2c, wc1a, wc1b, bc1c, wc2t, bc2c)

    return o1.reshape(B, C, H, W), o2.reshape(B, C, H, W)


# speedup vs baseline: 1.0792x; 1.0029x over previous
import functools

import jax
import jax.numpy as jnp
from jax.experimental import pallas as pl
from jax.experimental.pallas import tpu as pltpu

_BATCH_PER_STEP = 4


def _fused_kernel(x1_ref, x2_ref,
                  w1_ref, b1_ref, w2_ref, b2_ref,
                  wc1a_ref, wc1b_ref, bc1_ref, wc2_ref, bc2_ref,
                  o1_ref, o2_ref, *, n_j, lambda_c, lambda_s):
    C = x1_ref.shape[1]
    bf16 = jnp.bfloat16

    def _fold(a, op):
        w = a.shape[1]
        while w > 128:
            w //= 2
            a = op(a[:, :w], a[:, w:2 * w])
        return a

    for j in range(n_j):
        x1 = x1_ref[j]
        x2 = x2_ref[j]

        sum1 = jnp.sum(_fold(x1, jnp.add), axis=1, keepdims=True)
        sum2 = jnp.sum(_fold(x2, jnp.add), axis=1, keepdims=True)
        max1 = jnp.max(_fold(x1, jnp.maximum), axis=1, keepdims=True)
        max2 = jnp.max(_fold(x2, jnp.maximum), axis=1, keepdims=True)
        y = jnp.concatenate([sum1, sum2, max1, max2], axis=0)
        h = (jnp.dot(w1_ref[...], y.astype(bf16),
                     preferred_element_type=jnp.float32)
             + b1_ref[...])
        h = jnp.maximum(h, 0.0)
        z = jax.nn.sigmoid(
            jnp.dot(w2_ref[...], h.astype(bf16),
                    preferred_element_type=jnp.float32)
            + b2_ref[...])
        cw0 = lambda_c * z[0:C]
        cw1 = lambda_c * z[C:2 * C]

        hs = (jnp.dot(wc1a_ref[...], x1.astype(bf16),
                      preferred_element_type=jnp.float32)
              + jnp.dot(wc1b_ref[...], x2.astype(bf16),
                        preferred_element_type=jnp.float32)
              + bc1_ref[...])
        hs = jnp.maximum(hs, 0.0).astype(bf16)
        s = jax.nn.sigmoid(
            jnp.dot(wc2_ref[...], hs, preferred_element_type=jnp.float32)
            + bc2_ref[...])
        s0 = lambda_s * s[0:1]
        s1 = lambda_s * s[1:2]

        o1_ref[j] = x1 + (cw1 + s1) * x2
        o2_ref[j] = x2 + (cw0 + s0) * x1


def kernel(x1, x2, w1, b1, w2, b2, wc1, bc1, wc2, bc2):
    B, C, H, W = x1.shape
    HW = H * W
    lambda_c = 0.5
    lambda_s = 0.5
    bf16 = jnp.bfloat16
    x1r = x1.reshape(B, C, HW)
    x2r = x2.reshape(B, C, HW)

    n_j = _BATCH_PER_STEP if B % _BATCH_PER_STEP == 0 else 1
    n_steps = B // n_j

    hid_c = w1.shape[1]
    scale = jnp.concatenate([jnp.full((2 * C, 1), 1.0 / HW, jnp.float32),
                             jnp.ones((2 * C, 1), jnp.float32)], axis=0)
    w1t = (w1 * scale).T.astype(bf16)
    b1c = b1.reshape(hid_c, 1)

    w2t = w2.T.astype(bf16)
    b2c = b2.reshape(2 * C, 1)

    hid_s = wc1.shape[1]
    wc1a = wc1[0:C, :].T.astype(bf16)
    wc1b = wc1[C:2 * C, :].T.astype(bf16)
    bc1c = bc1.reshape(hid_s, 1)

    wc2t = wc2.T.astype(bf16)
    bc2c = bc2.reshape(2, 1)

    img_spec = pl.BlockSpec((n_j, C, HW), lambda b: (b, 0, 0))

    def const2d(shape):
        return pl.BlockSpec(shape, lambda b: (0, 0))

    o1, o2 = pl.pallas_call(
        functools.partial(_fused_kernel, n_j=n_j,
                          lambda_c=lambda_c, lambda_s=lambda_s),
        out_shape=(jax.ShapeDtypeStruct((B, C, HW), x1.dtype),
                   jax.ShapeDtypeStruct((B, C, HW), x1.dtype)),
        grid=(n_steps,),
        in_specs=[
            img_spec, img_spec,
            const2d((hid_c, 4 * C)), const2d((hid_c, 1)),
            const2d((2 * C, hid_c)), const2d((2 * C, 1)),
            const2d((hid_s, C)), const2d((hid_s, C)), const2d((hid_s, 1)),
            const2d((2, hid_s)), const2d((2, 1)),
        ],
        out_specs=[img_spec, img_spec],
        compiler_params=pltpu.CompilerParams(
            dimension_semantics=("parallel",)),
    )(x1r, x2r, w1t, b1c, w2t, b2c, wc1a, wc1b, bc1c, wc2t, bc2c)

    return o1.reshape(B, C, H, W), o2.reshape(B, C, H, W)
